# baseline (speedup 1.0000x reference)
import functools
import math

import jax
import jax.numpy as jnp
from jax import lax
from jax.experimental import pallas as pl
from jax.experimental.pallas import tpu as pltpu
from jax.experimental.pallas import tpu_sc as plsc

_D = 64
_SCALE = math.sqrt(_D)


def _gather_pairs(jidx, table_pairs):
    (B,) = jidx.shape
    info = plsc.get_sparse_core_info()
    nw = info.num_cores * info.num_subcores
    b_per_w = B // nw
    chunk = 256
    n_chunks = b_per_w // chunk

    mesh = plsc.VectorSubcoreMesh(core_axis_name="c", subcore_axis_name="s")

    @functools.partial(
        pl.kernel,
        out_type=jax.ShapeDtypeStruct((B, 2 * _D), jnp.float32),
        mesh=mesh,
        scratch_types=[
            pltpu.VMEM((b_per_w,), jnp.int32),
            pltpu.VMEM((chunk, 2 * _D), jnp.float32),
            pltpu.VMEM((chunk, 2 * _D), jnp.float32),
            pltpu.SemaphoreType.DMA,
            pltpu.SemaphoreType.DMA,
            pltpu.SemaphoreType.DMA,
            pltpu.SemaphoreType.DMA,
        ],
        compiler_params=pltpu.CompilerParams(use_tc_tiling_on_sc=True),
    )
    def gat(jidx_hbm, table_hbm, out_hbm, jidx_v, rows0, rows1,
            gsem0, gsem1, ssem0, ssem1):
        wid = lax.axis_index("s") * info.num_cores + lax.axis_index("c")
        base = wid * b_per_w
        bufs = ((rows0, gsem0, ssem0), (rows1, gsem1, ssem1))

        def gather(c, rows, sem):
            return pltpu.make_async_copy(
                table_hbm.at[jidx_v.at[pl.ds(c * chunk, chunk)]], rows, sem)

        def store(c, rows, sem):
            return pltpu.make_async_copy(
                rows, out_hbm.at[pl.ds(base + c * chunk, chunk)], sem)

        pltpu.sync_copy(jidx_hbm.at[pl.ds(base, b_per_w)], jidx_v)
        gather(0, rows0, gsem0).start()

        @pl.loop(0, n_chunks, step=2)
        def _(ci):
            for b in range(2):
                cur = ci + b
                rows, gsem, ssem = bufs[b]
                nrows, ngsem, nssem = bufs[1 - b]
                nxt = cur + 1

                @pl.when(nxt < n_chunks)
                def _():
                    @pl.when(nxt >= 2)
                    def _():
                        store(nxt - 2, nrows, nssem).wait()

                    gather(nxt, nrows, ngsem).start()

                gather(cur, rows, gsem).wait()
                store(cur, rows, ssem).start()

        store(n_chunks - 2, rows0, ssem0).wait()
        store(n_chunks - 1, rows1, ssem1).wait()

    return gat(jidx, table_pairs)


def _select_scale_t(pairs2, parity, S, Bt):
    blk = 512
    nb = Bt // blk

    def body(p_ref, par_ref, o_ref):
        x = p_ref[:, :]
        par = par_ref[:].reshape(blk, 1)
        lo = x[:, :_D]
        hi = x[:, _D:]
        sel = jnp.where(par > 0, hi, lo) * _SCALE
        o_ref[:, :] = sel.T

    return pl.pallas_call(
        body,
        grid=(S * nb,),
        in_specs=[
            pl.BlockSpec((blk, 2 * _D), lambda i: (i, 0)),
            pl.BlockSpec((blk,), lambda i: (i,)),
        ],
        out_specs=pl.BlockSpec((_D, blk), lambda i: (i // nb, i % nb)),
        out_shape=jax.ShapeDtypeStruct((S * _D, Bt), jnp.float32),
    )(pairs2, parity)


def kernel(input, table):
    b, s = input.shape
    idx_s = input.T.reshape(s * b).astype(jnp.int32)
    pairs = table.reshape(table.shape[0] // 2, 2 * _D)
    gathered = _gather_pairs(idx_s >> 1, pairs)
    out_t = _select_scale_t(gathered, idx_s & 1, s, b)
    return jnp.transpose(out_t.reshape(s, _D, b), (2, 0, 1))


# speedup vs baseline: 1.1244x; 1.1244x over previous
import functools
import math

import jax
import jax.numpy as jnp
from jax import lax
from jax.experimental import pallas as pl
from jax.experimental.pallas import tpu as pltpu
from jax.experimental.pallas import tpu_sc as plsc

_D = 64
_SCALE = math.sqrt(_D)


def _gather_pairs(jidx, table_pairs):
    (B,) = jidx.shape
    info = plsc.get_sparse_core_info()
    nw = info.num_cores * info.num_subcores
    b_per_w = B // nw
    chunk = 256
    n_chunks = b_per_w // chunk

    mesh = plsc.VectorSubcoreMesh(core_axis_name="c", subcore_axis_name="s")

    @functools.partial(
        pl.kernel,
        out_type=jax.ShapeDtypeStruct((B, 2 * _D), jnp.float32),
        mesh=mesh,
        scratch_types=[
            pltpu.VMEM((b_per_w,), jnp.int32),
            pltpu.VMEM((chunk, 2 * _D), jnp.float32),
            pltpu.VMEM((chunk, 2 * _D), jnp.float32),
            pltpu.SemaphoreType.DMA,
            pltpu.SemaphoreType.DMA,
            pltpu.SemaphoreType.DMA,
            pltpu.SemaphoreType.DMA,
        ],
        compiler_params=pltpu.CompilerParams(use_tc_tiling_on_sc=True),
    )
    def gat(jidx_hbm, table_hbm, out_hbm, jidx_v, rows0, rows1,
            gsem0, gsem1, ssem0, ssem1):
        wid = lax.axis_index("s") * info.num_cores + lax.axis_index("c")
        base = wid * b_per_w
        bufs = ((rows0, gsem0, ssem0), (rows1, gsem1, ssem1))

        def gather(c, rows, sem):
            return pltpu.make_async_copy(
                table_hbm.at[jidx_v.at[pl.ds(c * chunk, chunk)]], rows, sem)

        def store(c, rows, sem):
            return pltpu.make_async_copy(
                rows, out_hbm.at[pl.ds(base + c * chunk, chunk)], sem)

        pltpu.sync_copy(jidx_hbm.at[pl.ds(base, b_per_w)], jidx_v)
        gather(0, rows0, gsem0).start()

        @pl.loop(0, n_chunks, step=2)
        def _(ci):
            for b in range(2):
                cur = ci + b
                rows, gsem, ssem = bufs[b]
                nrows, ngsem, nssem = bufs[1 - b]
                nxt = cur + 1

                @pl.when(nxt < n_chunks)
                def _():
                    @pl.when(nxt >= 2)
                    def _():
                        store(nxt - 2, nrows, nssem).wait()

                    gather(nxt, nrows, ngsem).start()

                gather(cur, rows, gsem).wait()
                store(cur, rows, ssem).start()

        store(n_chunks - 2, rows0, ssem0).wait()
        store(n_chunks - 1, rows1, ssem1).wait()

    return gat(jidx, table_pairs)


def _select_scale(pairs2, parity):
    (B, _) = pairs2.shape
    blk = 1024

    def body(p_ref, par_ref, o_ref):
        x = p_ref[:, :]
        par = par_ref[:].reshape(blk, 1)
        lo = x[:, :_D]
        hi = x[:, _D:]
        o_ref[:, :] = jnp.where(par > 0, hi, lo) * _SCALE

    return pl.pallas_call(
        body,
        grid=(B // blk,),
        in_specs=[
            pl.BlockSpec((blk, 2 * _D), lambda i: (i, 0)),
            pl.BlockSpec((blk,), lambda i: (i,)),
        ],
        out_specs=pl.BlockSpec((blk, _D), lambda i: (i, 0)),
        out_shape=jax.ShapeDtypeStruct((B, _D), jnp.float32),
    )(pairs2, parity)


def kernel(input, table):
    b, s = input.shape
    idx = input.reshape(b * s).astype(jnp.int32)
    pairs = table.reshape(table.shape[0] // 2, 2 * _D)
    gathered = _gather_pairs(idx >> 1, pairs)
    out = _select_scale(gathered, idx & 1)
    return out.reshape(b, s, _D)


# speedup vs baseline: 1.6324x; 1.4518x over previous
import functools
import math

import jax
import jax.numpy as jnp
from jax import lax
from jax.experimental import pallas as pl
from jax.experimental.pallas import tpu as pltpu
from jax.experimental.pallas import tpu_sc as plsc

_D = 64
_SCALE = math.sqrt(_D)


def _embed(idx, table):
    (B,) = idx.shape
    info = plsc.get_sparse_core_info()
    nw = info.num_cores * info.num_subcores
    b_per_w = B // nw
    chunk = 512
    n_chunks = b_per_w // chunk

    mesh = plsc.VectorSubcoreMesh(core_axis_name="c", subcore_axis_name="s")

    @functools.partial(
        pl.kernel,
        out_type=jax.ShapeDtypeStruct((B, _D), jnp.float32),
        mesh=mesh,
        scratch_types=[
            pltpu.VMEM((b_per_w,), jnp.int32),
            pltpu.VMEM((chunk, _D), jnp.float32),
            pltpu.VMEM((chunk, _D), jnp.float32),
            pltpu.SemaphoreType.DMA,
            pltpu.SemaphoreType.DMA,
            pltpu.SemaphoreType.DMA,
            pltpu.SemaphoreType.DMA,
        ],
        compiler_params=pltpu.CompilerParams(use_tc_tiling_on_sc=False),
    )
    def emb(idx_hbm, table_hbm, out_hbm, idx_v, rows0, rows1,
            gsem0, gsem1, ssem0, ssem1):
        wid = lax.axis_index("s") * info.num_cores + lax.axis_index("c")
        base = wid * b_per_w
        bufs = ((rows0, gsem0, ssem0), (rows1, gsem1, ssem1))

        def gather(c, rows, sem):
            return pltpu.make_async_copy(
                table_hbm.at[idx_v.at[pl.ds(c * chunk, chunk)]], rows, sem)

        def store(c, rows, sem):
            return pltpu.make_async_copy(
                rows, out_hbm.at[pl.ds(base + c * chunk, chunk)], sem)

        pltpu.sync_copy(idx_hbm.at[pl.ds(base, b_per_w)], idx_v)
        gather(0, rows0, gsem0).start()

        @pl.loop(0, n_chunks, step=2)
        def _(ci):
            for b in range(2):
                cur = ci + b
                rows, gsem, ssem = bufs[b]
                nrows, ngsem, nssem = bufs[1 - b]
                nxt = cur + 1

                @pl.when(nxt < n_chunks)
                def _():
                    @pl.when(nxt >= 2)
                    def _():
                        store(nxt - 2, nrows, nssem).wait()

                    gather(nxt, nrows, ngsem).start()

                gather(cur, rows, gsem).wait()

                def scale_row(r, c2):
                    for j in range(_D // 16):
                        sl = pl.ds(j * 16, 16)
                        rows[r, sl] = rows[r, sl] * _SCALE
                    return c2

                lax.fori_loop(0, chunk, scale_row, 0, unroll=4)
                store(cur, rows, ssem).start()

        store(n_chunks - 2, rows0, ssem0).wait()
        store(n_chunks - 1, rows1, ssem1).wait()

    return emb(idx, table)


def kernel(input, table):
    b, s = input.shape
    idx = input.reshape(b * s).astype(jnp.int32)
    out = _embed(idx, table)
    return out.reshape(b, s, _D)
